# baseline (speedup 1.0000x reference)
import jax
import jax.numpy as jnp
from jax.experimental import pallas as pl
from jax.experimental.pallas import tpu as pltpu

_B, _T, _R, _C = 4, 8192, 129, 2
_TT, _TI = _T // 128, 128
_NP = 20
_PROT = _B * _NP


def _body(tok_ref, x_ref, o_ref, prot_ref, bulk_sems, prot_sems, out_sems):
    t = tok_ref[0]

    for b in range(_B):
        base = b * _R + _NP
        pltpu.make_async_copy(
            x_ref.at[pl.ds(base, 54)], o_ref.at[pl.ds(base, 54)],
            bulk_sems.at[2 * b],
        ).start()
        pltpu.make_async_copy(
            x_ref.at[pl.ds(base + 54, 55)], o_ref.at[pl.ds(base + 54, 55)],
            bulk_sems.at[2 * b + 1],
        ).start()
        pltpu.make_async_copy(
            x_ref.at[pl.ds(b * _R, _NP)], prot_ref.at[pl.ds(b * _NP, _NP)],
            prot_sems.at[b],
        ).start()

    for b in range(_B):
        pltpu.make_async_copy(
            x_ref.at[pl.ds(b * _R, _NP)], prot_ref.at[pl.ds(b * _NP, _NP)],
            prot_sems.at[b],
        ).wait()

    shape = (_PROT, _TT * _C, _TI)
    reg = jax.lax.broadcasted_iota(jnp.int32, shape, 0) % _NP
    col0 = (jax.lax.broadcasted_iota(jnp.int32, shape, 1) % _C) == 0

    is_start = t == 0
    is_digit = (t >= 1) & (t <= 10)
    is_plus = t == 11
    is_minus = t == 12
    is_equals = t == 13
    digit_val = (t - 1) % 10

    digit_band = (reg >= 2) & (reg <= 11) & col0
    digit_hit = (reg == 2 + (digit_val % 10)) & col0
    op_reg = (reg == 1) & col0
    result_regs = (reg >= 14) & (reg <= 16) & col0

    out = prot_ref[...]
    out = jnp.where(is_start, 0.0, out)
    out = jnp.where(is_digit & digit_band, 0.0, out)
    out = jnp.where(is_digit & digit_hit, 1.0, out)
    out = jnp.where(is_plus & op_reg, 1.0, out)
    out = jnp.where(is_minus & op_reg, -1.0, out)
    out = jnp.where(is_equals & (result_regs | op_reg | digit_band), 0.0, out)
    prot_ref[...] = out

    for b in range(_B):
        pltpu.make_async_copy(
            prot_ref.at[pl.ds(b * _NP, _NP)], o_ref.at[pl.ds(b * _R, _NP)],
            out_sems.at[b],
        ).start()
    for b in range(_B):
        pltpu.make_async_copy(
            prot_ref.at[pl.ds(b * _NP, _NP)], o_ref.at[pl.ds(b * _R, _NP)],
            out_sems.at[b],
        ).wait()
        base = b * _R + _NP
        pltpu.make_async_copy(
            x_ref.at[pl.ds(base, 54)], o_ref.at[pl.ds(base, 54)],
            bulk_sems.at[2 * b],
        ).wait()
        pltpu.make_async_copy(
            x_ref.at[pl.ds(base + 54, 55)], o_ref.at[pl.ds(base + 54, 55)],
            bulk_sems.at[2 * b + 1],
        ).wait()


def kernel(carrier_freq, src_token, tgt_token):
    x3 = (
        carrier_freq.transpose(0, 2, 1, 3)
        .reshape(_B, _R, _TT, _TI, _C)
        .transpose(0, 1, 2, 4, 3)
        .reshape(_B * _R, _TT * _C, _TI)
    )
    tok = jnp.asarray(src_token, jnp.int32).reshape(1)
    out = pl.pallas_call(
        _body,
        in_specs=[
            pl.BlockSpec(memory_space=pltpu.SMEM),
            pl.BlockSpec(memory_space=pltpu.MemorySpace.HBM),
        ],
        out_specs=pl.BlockSpec(memory_space=pltpu.MemorySpace.HBM),
        out_shape=jax.ShapeDtypeStruct((_B * _R, _TT * _C, _TI), jnp.float32),
        scratch_shapes=[
            pltpu.VMEM((_PROT, _TT * _C, _TI), jnp.float32),
            pltpu.SemaphoreType.DMA((2 * _B,)),
            pltpu.SemaphoreType.DMA((_B,)),
            pltpu.SemaphoreType.DMA((_B,)),
        ],
    )(tok, x3)
    return (
        out.reshape(_B, _R, _TT, _C, _TI)
        .transpose(0, 1, 2, 4, 3)
        .reshape(_B, _R, _T, _C)
        .transpose(0, 2, 1, 3)
    )


# speedup vs baseline: 34.8407x; 34.8407x over previous
import jax
import jax.numpy as jnp
from jax.experimental import pallas as pl
from jax.experimental.pallas import tpu as pltpu

_B, _T, _R, _C = 4, 8192, 129, 2
_TT, _TI = _T // 128, 128
_D1 = _TT * _C
_NP = 20
_BL = 64


def _reg_tables(src_token):
    t = jnp.asarray(src_token, jnp.int32)
    reg = jnp.arange(_NP, dtype=jnp.int32)

    is_start = t == 0
    is_digit = (t >= 1) & (t <= 10)
    is_plus = t == 11
    is_minus = t == 12
    is_equals = t == 13
    digit_val = (t - 1) % 10

    digit_band = (reg >= 2) & (reg <= 11)
    digit_hit = reg == 2 + (digit_val % 10)
    op_reg = reg == 1
    result_regs = (reg >= 14) & (reg <= 16)

    m0 = (is_start
          | (is_digit & digit_band)
          | ((is_plus | is_minus) & op_reg)
          | (is_equals & (result_regs | op_reg | digit_band)))
    m1 = jnp.broadcast_to(is_start, (_NP,))
    v = jnp.zeros((_NP,), jnp.float32)
    v = jnp.where(is_digit & digit_hit, 1.0, v)
    v = jnp.where(is_plus & op_reg, 1.0, v)
    v = jnp.where(is_minus & op_reg, -1.0, v)
    return m0.astype(jnp.int32), m1.astype(jnp.int32), v


def _body(m0_ref, m1_ref, v_ref, x_ref, o_ref):
    col0 = (jax.lax.broadcasted_iota(jnp.int32, (_BL, _TI), 0) % _C) == 0
    for r in range(_NP):
        cond = jnp.where(col0, m0_ref[r], m1_ref[r]) != 0
        x_r = x_ref[0, r]
        o_ref[0, r] = jnp.where(cond, v_ref[r], x_r)
    o_ref[0, _NP:] = x_ref[0, _NP:]


def kernel(carrier_freq, src_token, tgt_token):
    x4 = (
        carrier_freq.transpose(0, 2, 1, 3)
        .reshape(_B, _R, _TT, _TI, _C)
        .transpose(0, 1, 2, 4, 3)
        .reshape(_B, _R, _D1, _TI)
    )
    m0, m1, v = _reg_tables(src_token)
    out = pl.pallas_call(
        _body,
        grid=(_B, _D1 // _BL),
        in_specs=[
            pl.BlockSpec(memory_space=pltpu.SMEM),
            pl.BlockSpec(memory_space=pltpu.SMEM),
            pl.BlockSpec(memory_space=pltpu.SMEM),
            pl.BlockSpec((1, _R, _BL, _TI), lambda i, j: (i, 0, j, 0)),
        ],
        out_specs=pl.BlockSpec((1, _R, _BL, _TI), lambda i, j: (i, 0, j, 0)),
        out_shape=jax.ShapeDtypeStruct((_B, _R, _D1, _TI), jnp.float32),
        compiler_params=pltpu.CompilerParams(
            dimension_semantics=("parallel", "parallel"),
        ),
    )(m0, m1, v, x4)
    return (
        out.reshape(_B, _R, _TT, _C, _TI)
        .transpose(0, 1, 2, 4, 3)
        .reshape(_B, _R, _T, _C)
        .transpose(0, 2, 1, 3)
    )
